# baseline (speedup 1.0000x reference)
import functools

import jax
import jax.numpy as jnp
from jax import lax
from jax.experimental import pallas as pl
from jax.experimental.pallas import tpu as pltpu
from jax.experimental.pallas import tpu_sc as plsc

_D = 64
_H = 50
_B = 16384
_NBT = _B // 128
_BTW = 4
_RW = _BTW * 128
_OUT_FLAT = _B * _H * _D


@functools.partial(
    pl.kernel,
    out_type=jax.ShapeDtypeStruct((_OUT_FLAT,), jnp.float32),
    mesh=plsc.VectorSubcoreMesh(core_axis_name="c", subcore_axis_name="s"),
    scratch_types=[
        pltpu.VMEM((_H, 128), jnp.int32),
        pltpu.VMEM((_H, 128), jnp.int32),
        pltpu.VMEM((_H, 128), jnp.int32),
        pltpu.VMEM((_H, 128), jnp.int32),
        pltpu.VMEM((_RW, _D), jnp.float32),
        pltpu.VMEM((_RW, _D), jnp.float32),
        pltpu.VMEM((_D * _RW,), jnp.float32),
        pltpu.SemaphoreType.DMA,
        pltpu.SemaphoreType.DMA,
        pltpu.SemaphoreType.DMA,
    ],
    compiler_params=pltpu.CompilerParams(
        use_tc_tiling_on_sc=False, needs_layout_passes=False
    ),
)
def _gather_kernel(xt_hbm, table_hbm, out_hbm,
                   idx0, idx1, idx2, idx3, rows0, rows1, t_v,
                   g0, g1, osem):
    wid = lax.axis_index("s") * 2 + lax.axis_index("c")
    bt_base = wid * _BTW
    idxs = (idx0, idx1, idx2, idx3)
    rows = (rows0, rows1)
    gsems = (g0, g1)
    iota16 = lax.iota(jnp.int32, 16)

    for btl in range(_BTW):
        pltpu.sync_copy(
            xt_hbm.at[:, pl.ds(pl.multiple_of((bt_base + btl) * 128, 128), 128)],
            idxs[btl],
        )

    def fire_gathers(h, p):
        for btl in range(_BTW):
            pltpu.async_copy(
                table_hbm.at[idxs[btl].at[h]],
                rows[p].at[pl.ds(btl * 128, 128)],
                gsems[p],
            )

    def wait_gathers(p):
        pltpu.make_async_copy(
            table_hbm.at[pl.ds(0, _RW)], rows[p], gsems[p]
        ).wait()

    def wait_stores():
        pltpu.make_async_copy(
            t_v, out_hbm.at[pl.ds(0, _D * _RW)], osem
        ).wait()

    def transpose_rows(p):
        r = rows[p]

        def dbody(d, carry):
            dvec = jnp.full((16,), 0, jnp.int32) + d
            toff = (
                lax.shift_right_logical(d, 3) * (_BTW * 1024)
                + lax.bitwise_and(d, 7) * 128
            )
            for btl in range(_BTW):
                for jg in range(8):
                    vec = plsc.load_gather(
                        r, [iota16 + (btl * 128 + jg * 16), dvec]
                    )
                    t_v[pl.ds(toff + (btl * 1024 + jg * 16), 16)] = vec
            return carry

        lax.fori_loop(0, _D, dbody, 0)

    def fire_stores(h):
        for d8 in range(8):
            r0 = (h * 8 + d8) * _NBT + bt_base
            pltpu.async_copy(
                t_v.at[pl.ds(d8 * (_BTW * 1024), _BTW * 1024)],
                out_hbm.at[pl.ds(pl.multiple_of(r0 * 1024, 1024), _BTW * 1024)],
                osem,
            )

    fire_gathers(0, 0)

    def body(hh, carry):
        for p in range(2):
            h = hh * 2 + p
            if p == 0:
                fire_gathers(h + 1, 1)
            else:
                @pl.when(hh < _H // 2 - 1)
                def _():
                    fire_gathers(h + 1, 0)
            wait_gathers(p)
            if p == 0:
                @pl.when(hh >= 1)
                def _():
                    wait_stores()
            else:
                wait_stores()
            transpose_rows(p)
            fire_stores(h)
        return carry

    lax.fori_loop(0, _H // 2, body, 0)
    wait_stores()


def kernel(x, embeddings):
    xt = x.T.astype(jnp.int32)
    out = _gather_kernel(xt, embeddings)
    out5 = out.reshape(_H, 8, _NBT, 8, 128)
    return out5.transpose(2, 4, 0, 1, 3).reshape(_B, _H, _D)


# speedup vs baseline: 1.4355x; 1.4355x over previous
import functools

import jax
import jax.numpy as jnp
from jax import lax
from jax.experimental import pallas as pl
from jax.experimental.pallas import tpu as pltpu
from jax.experimental.pallas import tpu_sc as plsc

_D = 64
_H = 50
_B = 16384
_NBT = _B // 128
_BTW = 4
_RW = _BTW * 128
_OUT_FLAT = _B * _H * _D


@functools.partial(
    pl.kernel,
    out_type=jax.ShapeDtypeStruct((_OUT_FLAT,), jnp.float32),
    mesh=plsc.VectorSubcoreMesh(core_axis_name="c", subcore_axis_name="s"),
    scratch_types=[
        pltpu.VMEM((_H, 128), jnp.int32),
        pltpu.VMEM((_H, 128), jnp.int32),
        pltpu.VMEM((_H, 128), jnp.int32),
        pltpu.VMEM((_H, 128), jnp.int32),
        pltpu.VMEM((_RW, _D), jnp.float32),
        pltpu.VMEM((_RW, _D), jnp.float32),
        pltpu.VMEM((_D * _RW,), jnp.float32),
        pltpu.SemaphoreType.DMA,
        pltpu.SemaphoreType.DMA,
        pltpu.SemaphoreType.DMA,
    ],
    compiler_params=pltpu.CompilerParams(
        use_tc_tiling_on_sc=False, needs_layout_passes=False
    ),
)
def _gather_kernel(xt_hbm, table_hbm, out_hbm,
                   idx0, idx1, idx2, idx3, rows0, rows1, t_v,
                   g0, g1, osem):
    wid = lax.axis_index("s") * 2 + lax.axis_index("c")
    bt_base = wid * _BTW
    idxs = (idx0, idx1, idx2, idx3)
    rows = (rows0, rows1)
    gsems = (g0, g1)
    iota16 = lax.iota(jnp.int32, 16)

    for btl in range(_BTW):
        pltpu.sync_copy(
            xt_hbm.at[:, pl.ds(pl.multiple_of((bt_base + btl) * 128, 128), 128)],
            idxs[btl],
        )

    def fire_gathers(h, p):
        for btl in range(_BTW):
            pltpu.async_copy(
                table_hbm.at[idxs[btl].at[h]],
                rows[p].at[pl.ds(btl * 128, 128)],
                gsems[p],
            )

    def wait_gathers(p):
        pltpu.make_async_copy(
            table_hbm.at[pl.ds(0, _RW)], rows[p], gsems[p]
        ).wait()

    def wait_stores():
        pltpu.make_async_copy(
            t_v, out_hbm.at[pl.ds(0, _D * _RW)], osem
        ).wait()

    def transpose_rows(p):
        r = rows[p]

        @plsc.parallel_loop(0, _D, unroll=4)
        def dbody(d):
            dvec = jnp.full((16,), 0, jnp.int32) + d
            toff = (
                lax.shift_right_logical(d, 3) * (_BTW * 1024)
                + lax.bitwise_and(d, 7) * 128
            )
            for btl in range(_BTW):
                for jg in range(8):
                    vec = plsc.load_gather(
                        r, [iota16 + (btl * 128 + jg * 16), dvec]
                    )
                    t_v[pl.ds(toff + (btl * 1024 + jg * 16), 16)] = vec

    def fire_stores(h):
        for d8 in range(8):
            r0 = (h * 8 + d8) * _NBT + bt_base
            pltpu.async_copy(
                t_v.at[pl.ds(d8 * (_BTW * 1024), _BTW * 1024)],
                out_hbm.at[pl.ds(pl.multiple_of(r0 * 1024, 1024), _BTW * 1024)],
                osem,
            )

    fire_gathers(0, 0)

    def body(hh, carry):
        for p in range(2):
            h = hh * 2 + p
            if p == 0:
                fire_gathers(h + 1, 1)
            else:
                @pl.when(hh < _H // 2 - 1)
                def _():
                    fire_gathers(h + 1, 0)
            wait_gathers(p)
            if p == 0:
                @pl.when(hh >= 1)
                def _():
                    wait_stores()
            else:
                wait_stores()
            transpose_rows(p)
            fire_stores(h)
        return carry

    lax.fori_loop(0, _H // 2, body, 0)
    wait_stores()


def kernel(x, embeddings):
    xt = x.T.astype(jnp.int32)
    out = _gather_kernel(xt, embeddings)
    out5 = out.reshape(_H, 8, _NBT, 8, 128)
    return out5.transpose(2, 4, 0, 1, 3).reshape(_B, _H, _D)


# speedup vs baseline: 1.4968x; 1.0427x over previous
import functools

import jax
import jax.numpy as jnp
from jax import lax
from jax.experimental import pallas as pl
from jax.experimental.pallas import tpu as pltpu
from jax.experimental.pallas import tpu_sc as plsc

_D = 64
_B_TOTAL = 16384 * 50
_NC = 2
_NS = 16
_NW = _NC * _NS
_B_PER_W = _B_TOTAL // _NW
_IDXW = 128
_K = 4
_R = _IDXW * _K
_NCHUNK = _B_PER_W // _R


@functools.partial(
    pl.kernel,
    out_type=jax.ShapeDtypeStruct((_B_TOTAL, _D), jnp.float32),
    mesh=plsc.VectorSubcoreMesh(core_axis_name="c", subcore_axis_name="s"),
    scratch_types=[
        pltpu.VMEM((_B_PER_W,), jnp.int32),
        pltpu.VMEM((2 * _R, _D), jnp.float32),
        pltpu.SemaphoreType.DMA,
        pltpu.SemaphoreType.DMA,
        pltpu.SemaphoreType.DMA,
        pltpu.SemaphoreType.DMA,
    ],
    compiler_params=pltpu.CompilerParams(use_tc_tiling_on_sc=False),
)
def _gather_kernel(idx_hbm, table_hbm, out_hbm, idx_v, rows_v, g0, g1, o0, o1):
    wid = lax.axis_index("s") * _NC + lax.axis_index("c")
    row_base = pl.multiple_of(wid * _B_PER_W, _B_PER_W)

    pltpu.sync_copy(idx_hbm.at[pl.ds(row_base, _B_PER_W)], idx_v)

    gsems = (g0, g1)
    osems = (o0, o1)

    def fire_gathers(g, b, gsem):
        base = pl.multiple_of(g * _R, _R)
        for j in range(_K):
            pltpu.async_copy(
                table_hbm.at[idx_v.at[pl.ds(base + j * _IDXW, _IDXW)]],
                rows_v.at[pl.ds(b * _R + j * _IDXW, _IDXW)],
                gsem,
            )

    def wait_gathers(b, gsem):
        pltpu.make_async_copy(
            out_hbm.at[pl.ds(0, _R)],
            rows_v.at[pl.ds(b * _R, _R)],
            gsem,
        ).wait()

    def fire_store(g, b, osem):
        off = pl.multiple_of(row_base + g * _R, _R)
        pltpu.async_copy(
            rows_v.at[pl.ds(b * _R, _R)],
            out_hbm.at[pl.ds(off, _R)],
            osem,
        )

    def wait_store(b, osem):
        pltpu.make_async_copy(
            rows_v.at[pl.ds(b * _R, _R)],
            out_hbm.at[pl.ds(0, _R)],
            osem,
        ).wait()

    def body(i, carry):
        for b in range(2):
            g = i * 2 + b
            @pl.when(i >= 1)
            def _():
                wait_store(b, osems[b])
            fire_gathers(g, b, gsems[b])
        for b in range(2):
            g = i * 2 + b
            wait_gathers(b, gsems[b])
            fire_store(g, b, osems[b])
        return carry

    lax.fori_loop(0, _NCHUNK // 2, body, 0)
    wait_store(0, o0)
    wait_store(1, o1)


def kernel(x, embeddings):
    idx = x.reshape(_B_TOTAL).astype(jnp.int32)
    out = _gather_kernel(idx, embeddings)
    return out.reshape(x.shape[0], x.shape[1], _D)
